# baseline (speedup 1.0000x reference)
import functools

import jax
import jax.numpy as jnp
from jax import lax
from jax.experimental import pallas as pl
from jax.experimental.pallas import tpu as pltpu
from jax.experimental.pallas import tpu_sc as plsc

_D = 32
_B = 16384
_UREG = 0.0025
_PREG = 0.0025
_NREG = 0.00025

_NC = 2
_NS = 16
_NW = _NC * _NS
_BPW = _B // _NW
_CH = 128
_NCH = _BPW // _CH
_NG = _BPW // 16


def _sc_gather_score(user_ids, pos_ids, neg_ids, user_t, item_t):
    mesh = plsc.VectorSubcoreMesh(core_axis_name="c", subcore_axis_name="s",
                                  num_cores=_NC, num_subcores=_NS)
    out_type = (
        jax.ShapeDtypeStruct((_B,), jnp.float32),
        jax.ShapeDtypeStruct((_B,), jnp.float32),
        jax.ShapeDtypeStruct((_NW, 16), jnp.float32),
    )
    scratch = [
        pltpu.VMEM((_NCH, _CH), jnp.int32),
        pltpu.VMEM((_NCH, _CH), jnp.int32),
        pltpu.VMEM((_NCH, _CH), jnp.int32),
        pltpu.VMEM((_D, _BPW), jnp.float32),
        pltpu.VMEM((_D, _BPW), jnp.float32),
        pltpu.VMEM((_D, _BPW), jnp.float32),
        pltpu.VMEM((_BPW,), jnp.float32),
        pltpu.VMEM((_BPW,), jnp.float32),
        pltpu.VMEM((16,), jnp.float32),
        pltpu.SemaphoreType.DMA,
    ]

    @functools.partial(pl.kernel, mesh=mesh, out_type=out_type,
                       scratch_types=scratch,
                       compiler_params=pltpu.CompilerParams(
                           needs_layout_passes=False,
                           use_tc_tiling_on_sc=False))
    def k(uids, pids, nids, utab, itab, posd_out, negd_out, reg_out,
          uidx, pidx, nidx, urows, prows, nrows, posd, negd, regv, sem):
        wid = lax.axis_index("s") * _NC + lax.axis_index("c")

        pltpu.sync_copy(uids.at[wid], uidx)
        pltpu.sync_copy(pids.at[wid], pidx)
        pltpu.sync_copy(nids.at[wid], nidx)

        def fire(d, _):
            copies = []
            for j in range(_NCH):
                sl = pl.ds(j * _CH, _CH)
                copies.append(pltpu.async_copy(
                    utab.at[d].at[uidx.at[j]], urows.at[d, sl], sem))
                copies.append(pltpu.async_copy(
                    itab.at[d].at[pidx.at[j]], prows.at[d, sl], sem))
                copies.append(pltpu.async_copy(
                    itab.at[d].at[nidx.at[j]], nrows.at[d, sl], sem))
            for c in copies:
                c.wait()
            return 0

        lax.fori_loop(0, _D, fire, 0)

        def group(g, reg_acc):
            sl = pl.ds(g * 16, 16)
            pos_acc = jnp.zeros((16,), jnp.float32)
            neg_acc = jnp.zeros((16,), jnp.float32)
            for d in range(_D):
                u = urows[d, sl]
                p = prows[d, sl]
                q = nrows[d, sl]
                pos_acc = pos_acc + u * p
                neg_acc = neg_acc + u * q
                reg_acc = reg_acc + (_UREG * (u * u) + _PREG * (p * p)
                                     + _NREG * (q * q))
            posd[sl] = pos_acc
            negd[sl] = neg_acc
            return reg_acc

        reg_acc = lax.fori_loop(0, _NG, group, jnp.zeros((16,), jnp.float32))
        regv[...] = reg_acc

        pltpu.sync_copy(posd, posd_out.at[pl.ds(wid * _BPW, _BPW)])
        pltpu.sync_copy(negd, negd_out.at[pl.ds(wid * _BPW, _BPW)])
        pltpu.sync_copy(regv, reg_out.at[wid])

    return k(user_ids.reshape(_NW, _NCH, _CH),
             pos_ids.reshape(_NW, _NCH, _CH),
             neg_ids.reshape(_NW, _NCH, _CH),
             user_t, item_t)


def _finalize_body(pos_ref, neg_ref, reg_ref, out_ref):
    x = neg_ref[...] - pos_ref[...]
    sp = jnp.maximum(x, 0.0) + jnp.log(1.0 + jnp.exp(-jnp.abs(x)))
    out_ref[...] = (jnp.sum(sp, keepdims=True) / _B
                    + jnp.sum(reg_ref[...], keepdims=True))


def _finalize(posd, negd, regp):
    out = pl.pallas_call(
        _finalize_body,
        out_shape=jax.ShapeDtypeStruct((1, 1), jnp.float32),
    )(posd.reshape(128, 128), negd.reshape(128, 128), regp.reshape(4, 128))
    return out[0, 0]


def kernel(user_ids, pos_ids, neg_ids, user_table, item_table):
    posd, negd, regp = _sc_gather_score(user_ids, pos_ids, neg_ids,
                                        user_table.T, item_table.T)
    return _finalize(posd, negd, regp)


# speedup vs baseline: 11.0929x; 11.0929x over previous
import functools

import jax
import jax.numpy as jnp
from jax import lax
from jax.experimental import pallas as pl
from jax.experimental.pallas import tpu as pltpu
from jax.experimental.pallas import tpu_sc as plsc

_D = 32
_B = 16384
_UREG = 0.0025
_PREG = 0.0025
_NREG = 0.00025

_NC = 2
_NS = 16
_NW = _NC * _NS
_BPW = _B // _NW
_RING = 4
_PASS = 256


def _sc_gather_score(user_ids, pos_ids, neg_ids, user_t, item_t):
    mesh = plsc.VectorSubcoreMesh(core_axis_name="c", subcore_axis_name="s",
                                  num_cores=_NC, num_subcores=_NS)
    out_type = (
        jax.ShapeDtypeStruct((_B,), jnp.float32),
        jax.ShapeDtypeStruct((_B,), jnp.float32),
        jax.ShapeDtypeStruct((_NW, 16), jnp.float32),
    )
    scratch = [
        pltpu.VMEM((32, 16), jnp.int32),
        pltpu.VMEM((_RING, _D, 128), jnp.float32),
        pltpu.VMEM((_PASS, _D), jnp.float32),
        pltpu.VMEM((_PASS, _D), jnp.float32),
        pltpu.VMEM((_PASS, _D), jnp.float32),
        pltpu.VMEM((_BPW,), jnp.float32),
        pltpu.VMEM((_BPW,), jnp.float32),
        pltpu.VMEM((16,), jnp.float32),
    ] + [pltpu.SemaphoreType.DMA] * _RING

    @functools.partial(pl.kernel, mesh=mesh, out_type=out_type,
                       scratch_types=scratch,
                       compiler_params=pltpu.CompilerParams(
                           needs_layout_passes=False))
    def k(uids, pids, nids, utab, itab, posd_out, negd_out, reg_out,
          vidx, ring, urows, prows, nrows, posd, negd, regv, *sems):
        wid = lax.axis_index("s") * _NC + lax.axis_index("c")
        lanes = lax.iota(jnp.int32, 16)

        def fetch_phase(ids_hbm, tab, rows, base):
            pltpu.sync_copy(ids_hbm.at[wid], vidx)

            def col_of(i):
                gi = base + i
                v = vidx[gi // 16]
                r = jnp.sum(jnp.where(lanes == lax.rem(gi, 16), v, 0))
                rbase = pl.multiple_of((r // 128) * 128, 128)
                return r, rbase

            def fire(i, b):
                _, rbase = col_of(i)
                pltpu.async_copy(tab.at[:, pl.ds(rbase, 128)],
                                 ring.at[b], sems[b])

            def wait(b):
                pltpu.make_async_copy(tab.at[:, pl.ds(0, 128)],
                                      ring.at[b], sems[b]).wait()

            def extract(i, b):
                r, _ = col_of(i)
                lane = lax.rem(r, 128)
                bb = jnp.full((16,), b, jnp.int32)
                ll = jnp.full((16,), lane, jnp.int32)
                lo = plsc.load_gather(ring, [bb, lanes, ll])
                hi = plsc.load_gather(ring, [bb, lanes + 16, ll])
                rows[i, pl.ds(0, 16)] = lo
                rows[i, pl.ds(16, 16)] = hi

            for b in range(_RING):
                fire(b, b)

            def body(blk, _):
                for b in range(_RING):
                    i = blk * _RING + b
                    wait(b)
                    extract(i, b)
                    fire(i + _RING, b)
                return 0

            lax.fori_loop(0, _PASS // _RING - 1, body, 0)
            for b in range(_RING):
                i = _PASS - _RING + b
                wait(b)
                extract(i, b)

        def group(g, reg_acc, base):
            rows16 = g * 16 + lanes
            pos_acc = jnp.zeros((16,), jnp.float32)
            neg_acc = jnp.zeros((16,), jnp.float32)
            for d in range(_D):
                col = jnp.full((16,), d, jnp.int32)
                u = plsc.load_gather(urows, [rows16, col])
                p = plsc.load_gather(prows, [rows16, col])
                q = plsc.load_gather(nrows, [rows16, col])
                pos_acc = pos_acc + u * p
                neg_acc = neg_acc + u * q
                reg_acc = reg_acc + (_UREG * (u * u) + _PREG * (p * p)
                                     + _NREG * (q * q))
            posd[pl.ds(base + g * 16, 16)] = pos_acc
            negd[pl.ds(base + g * 16, 16)] = neg_acc
            return reg_acc

        reg_acc = jnp.zeros((16,), jnp.float32)
        for base in range(0, _BPW, _PASS):
            fetch_phase(uids, utab, urows, base)
            fetch_phase(pids, itab, prows, base)
            fetch_phase(nids, itab, nrows, base)
            reg_acc = lax.fori_loop(
                0, _PASS // 16,
                functools.partial(lambda base, g, acc: group(g, acc, base),
                                  base),
                reg_acc)
        regv[...] = reg_acc

        pltpu.sync_copy(posd, posd_out.at[pl.ds(wid * _BPW, _BPW)])
        pltpu.sync_copy(negd, negd_out.at[pl.ds(wid * _BPW, _BPW)])
        pltpu.sync_copy(regv, reg_out.at[wid])

    return k(user_ids.reshape(_NW, 32, 16),
             pos_ids.reshape(_NW, 32, 16),
             neg_ids.reshape(_NW, 32, 16),
             user_t, item_t)


def _finalize_body(pos_ref, neg_ref, reg_ref, out_ref):
    x = neg_ref[...] - pos_ref[...]
    sp = jnp.maximum(x, 0.0) + jnp.log(1.0 + jnp.exp(-jnp.abs(x)))
    out_ref[...] = (jnp.sum(sp, keepdims=True) / _B
                    + jnp.sum(reg_ref[...], keepdims=True))


def _finalize(posd, negd, regp):
    out = pl.pallas_call(
        _finalize_body,
        out_shape=jax.ShapeDtypeStruct((1, 1), jnp.float32),
    )(posd.reshape(128, 128), negd.reshape(128, 128), regp.reshape(4, 128))
    return out[0, 0]


def kernel(user_ids, pos_ids, neg_ids, user_table, item_table):
    posd, negd, regp = _sc_gather_score(user_ids, pos_ids, neg_ids,
                                        user_table.T, item_table.T)
    return _finalize(posd, negd, regp)


# speedup vs baseline: 14.0275x; 1.2645x over previous
import functools

import jax
import jax.numpy as jnp
from jax import lax
from jax.experimental import pallas as pl
from jax.experimental.pallas import tpu as pltpu
from jax.experimental.pallas import tpu_sc as plsc

_D = 32
_B = 16384
_UREG = 0.0025
_PREG = 0.0025
_NREG = 0.00025

_NC = 2
_NS = 16
_NW = _NC * _NS
_BPW = _B // _NW
_RING = 8
_PASS = 128


def _sc_gather_score(user_ids, pos_ids, neg_ids, user_t, item_t):
    mesh = plsc.VectorSubcoreMesh(core_axis_name="c", subcore_axis_name="s",
                                  num_cores=_NC, num_subcores=_NS)
    out_type = (
        jax.ShapeDtypeStruct((_B,), jnp.float32),
        jax.ShapeDtypeStruct((_B,), jnp.float32),
        jax.ShapeDtypeStruct((_NW, 16), jnp.float32),
    )
    scratch = [
        pltpu.VMEM((32, 16), jnp.int32),
        pltpu.VMEM((_RING, _D, 128), jnp.float32),
        pltpu.VMEM((_PASS, _D), jnp.float32),
        pltpu.VMEM((_PASS, _D), jnp.float32),
        pltpu.VMEM((_PASS, _D), jnp.float32),
        pltpu.VMEM((_BPW,), jnp.float32),
        pltpu.VMEM((_BPW,), jnp.float32),
        pltpu.VMEM((16,), jnp.float32),
    ] + [pltpu.SemaphoreType.DMA] * _RING

    @functools.partial(pl.kernel, mesh=mesh, out_type=out_type,
                       scratch_types=scratch,
                       compiler_params=pltpu.CompilerParams(
                           needs_layout_passes=False))
    def k(uids, pids, nids, utab, itab, posd_out, negd_out, reg_out,
          vidx, ring, urows, prows, nrows, posd, negd, regv, *sems):
        wid = lax.axis_index("s") * _NC + lax.axis_index("c")
        lanes = lax.iota(jnp.int32, 16)

        def fetch_phase(ids_hbm, tab, rows, base):
            pltpu.sync_copy(ids_hbm.at[wid], vidx)

            def col_of(i):
                gi = base + i
                v = vidx[gi // 16]
                r = jnp.sum(jnp.where(lanes == lax.rem(gi, 16), v, 0))
                rbase = pl.multiple_of((r // 128) * 128, 128)
                return r, rbase

            def fire(i, b):
                _, rbase = col_of(i)
                pltpu.async_copy(tab.at[:, pl.ds(rbase, 128)],
                                 ring.at[b], sems[b])

            def wait(b):
                pltpu.make_async_copy(tab.at[:, pl.ds(0, 128)],
                                      ring.at[b], sems[b]).wait()

            def extract(i, b):
                r, _ = col_of(i)
                lane = lax.rem(r, 128)
                bb = jnp.full((16,), b, jnp.int32)
                ll = jnp.full((16,), lane, jnp.int32)
                lo = plsc.load_gather(ring, [bb, lanes, ll])
                hi = plsc.load_gather(ring, [bb, lanes + 16, ll])
                rows[i, pl.ds(0, 16)] = lo
                rows[i, pl.ds(16, 16)] = hi

            for b in range(_RING):
                fire(b, b)

            def body(blk, _):
                for b in range(_RING):
                    i = blk * _RING + b
                    wait(b)
                    extract(i, b)
                    fire(i + _RING, b)
                return 0

            lax.fori_loop(0, _PASS // _RING - 1, body, 0)
            for b in range(_RING):
                i = _PASS - _RING + b
                wait(b)
                extract(i, b)

        def group(g, reg_acc, base):
            rows16 = g * 16 + lanes
            pos_acc = jnp.zeros((16,), jnp.float32)
            neg_acc = jnp.zeros((16,), jnp.float32)
            for d in range(_D):
                col = jnp.full((16,), d, jnp.int32)
                u = plsc.load_gather(urows, [rows16, col])
                p = plsc.load_gather(prows, [rows16, col])
                q = plsc.load_gather(nrows, [rows16, col])
                pos_acc = pos_acc + u * p
                neg_acc = neg_acc + u * q
                reg_acc = reg_acc + (_UREG * (u * u) + _PREG * (p * p)
                                     + _NREG * (q * q))
            posd[pl.ds(base + g * 16, 16)] = pos_acc
            negd[pl.ds(base + g * 16, 16)] = neg_acc
            return reg_acc

        def one_pass(pi, reg_acc):
            base = pi * _PASS
            fetch_phase(uids, utab, urows, base)
            fetch_phase(pids, itab, prows, base)
            fetch_phase(nids, itab, nrows, base)
            return lax.fori_loop(
                0, _PASS // 16, lambda g, acc: group(g, acc, base), reg_acc)

        reg_acc = lax.fori_loop(0, _BPW // _PASS, one_pass,
                                jnp.zeros((16,), jnp.float32))
        regv[...] = reg_acc

        pltpu.sync_copy(posd, posd_out.at[pl.ds(wid * _BPW, _BPW)])
        pltpu.sync_copy(negd, negd_out.at[pl.ds(wid * _BPW, _BPW)])
        pltpu.sync_copy(regv, reg_out.at[wid])

    return k(user_ids.reshape(_NW, 32, 16),
             pos_ids.reshape(_NW, 32, 16),
             neg_ids.reshape(_NW, 32, 16),
             user_t, item_t)


def _finalize_body(pos_ref, neg_ref, reg_ref, out_ref):
    x = neg_ref[...] - pos_ref[...]
    sp = jnp.maximum(x, 0.0) + jnp.log(1.0 + jnp.exp(-jnp.abs(x)))
    out_ref[...] = (jnp.sum(sp, keepdims=True) / _B
                    + jnp.sum(reg_ref[...], keepdims=True))


def _finalize(posd, negd, regp):
    out = pl.pallas_call(
        _finalize_body,
        out_shape=jax.ShapeDtypeStruct((1, 1), jnp.float32),
    )(posd.reshape(128, 128), negd.reshape(128, 128), regp.reshape(4, 128))
    return out[0, 0]


def kernel(user_ids, pos_ids, neg_ids, user_table, item_table):
    posd, negd, regp = _sc_gather_score(user_ids, pos_ids, neg_ids,
                                        user_table.T, item_table.T)
    return _finalize(posd, negd, regp)
